# Initial kernel scaffold; baseline (speedup 1.0000x reference)
import jax
import jax.numpy as jnp
from jax.experimental import pallas as pl


def kernel(z, constellation):
    raise NotImplementedError("write your pallas kernel here")


# speedup vs baseline: 5.4451x; 5.4451x over previous
import functools

import jax
import jax.numpy as jnp
from jax import lax
from jax.experimental import pallas as pl
from jax.experimental.pallas import tpu as pltpu
from jax.experimental.pallas import tpu_sc as plsc


def _make_psk_kernel(n, m):
    info = plsc.get_sparse_core_info()
    nc, ns, lanes = info.num_cores, info.num_subcores, info.num_lanes
    nw = nc * ns
    assert lanes == 16
    assert n % nw == 0
    per_w = n // nw
    chunk = per_w
    for cand_chunk in (12800, 6400, 3200, 1600, 800, 400):
        if per_w % cand_chunk == 0:
            chunk = cand_chunk
            break
    nchunk = per_w // chunk
    vregs = chunk // lanes

    mesh = plsc.VectorSubcoreMesh(core_axis_name="c", subcore_axis_name="s")

    @functools.partial(
        pl.kernel,
        mesh=mesh,
        out_type=jax.ShapeDtypeStruct((2 * n,), jnp.float32),
        scratch_types=[
            pltpu.VMEM((2 * m,), jnp.float32),
            pltpu.VMEM((chunk,), jnp.int32),
            pltpu.VMEM((2 * chunk,), jnp.float32),
        ],
        compiler_params=pltpu.CompilerParams(needs_layout_passes=False),
    )
    def psk(z_hbm, tab_hbm, out_hbm, tab_v, z_v, out_v):
        wid = lax.axis_index("s") * nc + lax.axis_index("c")
        pltpu.sync_copy(tab_hbm, tab_v)
        iota2 = lax.iota(jnp.int32, lanes) * 2

        def chunk_body(k, _):
            base = wid * per_w + k * chunk

            pltpu.sync_copy(z_hbm.at[pl.ds(base, chunk)], z_v)

            def vreg_body(i, _):
                z16 = z_v[pl.ds(i * lanes, lanes)]
                t = z16 * 2
                c = plsc.load_gather(tab_v, [t])
                s = plsc.load_gather(tab_v, [t + 1])
                oidx = iota2 + i * (2 * lanes)
                plsc.store_scatter(out_v, [oidx], c)
                plsc.store_scatter(out_v, [oidx + 1], s)
                return 0

            lax.fori_loop(0, vregs, vreg_body, 0)
            pltpu.sync_copy(out_v, out_hbm.at[pl.ds(2 * base, 2 * chunk)])
            return 0

        lax.fori_loop(0, nchunk, chunk_body, 0)

    return psk


def kernel(z, constellation):
    b, l = z.shape
    m = constellation.shape[0]
    n = b * l
    zf = z.reshape(n).astype(jnp.int32)
    tabf = constellation.reshape(2 * m).astype(jnp.float32)
    out = _make_psk_kernel(n, m)(zf, tabf)
    return out.reshape(b, l, 2)


# speedup vs baseline: 7.0532x; 1.2953x over previous
import functools

import jax
import jax.numpy as jnp
from jax import lax
from jax.experimental import pallas as pl
from jax.experimental.pallas import tpu as pltpu
from jax.experimental.pallas import tpu_sc as plsc


def _make_psk_kernel(b, l, m):
    info = plsc.get_sparse_core_info()
    nc, ns, lanes = info.num_cores, info.num_subcores, info.num_lanes
    nw = nc * ns
    assert lanes == 16
    assert b % nw == 0
    rows_w = b // nw
    rchunk = rows_w
    for cand in (64, 32, 16):
        if rows_w % cand == 0:
            rchunk = cand
            break
    nchunk = rows_w // rchunk
    groups = rchunk // lanes

    mesh = plsc.VectorSubcoreMesh(core_axis_name="c", subcore_axis_name="s")

    @functools.partial(
        pl.kernel,
        mesh=mesh,
        out_type=jax.ShapeDtypeStruct((b, l, 2), jnp.float32),
        scratch_types=[
            pltpu.VMEM((m, 2), jnp.float32),
            pltpu.VMEM((rchunk, l), jnp.int32),
            pltpu.VMEM((rchunk, l, 2), jnp.float32),
        ],
        compiler_params=pltpu.CompilerParams(
            needs_layout_passes=False, use_tc_tiling_on_sc=False
        ),
    )
    def psk(z_hbm, tab_hbm, out_hbm, tab_v, z_v, out_v):
        wid = lax.axis_index("s") * nc + lax.axis_index("c")
        pltpu.sync_copy(tab_hbm, tab_v)
        iota = lax.iota(jnp.int32, lanes)
        k0 = jnp.zeros((lanes,), jnp.int32)
        k1 = jnp.ones((lanes,), jnp.int32)
        rvecs = [iota + g * lanes for g in range(groups)]

        def chunk_body(k, _):
            row0 = wid * rows_w + k * rchunk
            pltpu.sync_copy(z_hbm.at[pl.ds(row0, rchunk)], z_v)

            def col_body(c, _):
                cv = jnp.full((lanes,), 0, jnp.int32) + c
                for g in range(groups):
                    rv = rvecs[g]
                    z16 = plsc.load_gather(z_v, [rv, cv])
                    cc = plsc.load_gather(tab_v, [z16, k0])
                    ss = plsc.load_gather(tab_v, [z16, k1])
                    plsc.store_scatter(out_v, [rv, cv, k0], cc)
                    plsc.store_scatter(out_v, [rv, cv, k1], ss)
                return 0

            lax.fori_loop(0, l, col_body, 0)
            pltpu.sync_copy(out_v, out_hbm.at[pl.ds(row0, rchunk)])
            return 0

        lax.fori_loop(0, nchunk, chunk_body, 0)

    return psk


def kernel(z, constellation):
    b, l = z.shape
    m = constellation.shape[0]
    zz = z.astype(jnp.int32)
    tab = constellation.astype(jnp.float32)
    return _make_psk_kernel(b, l, m)(zz, tab)


# speedup vs baseline: 94.7852x; 13.4386x over previous
import functools

import jax
import jax.numpy as jnp
from jax import lax
from jax.experimental import pallas as pl
from jax.experimental.pallas import tpu as pltpu
from jax.experimental.pallas import tpu_sc as plsc


def _make_psk_kernel(b, l, m):
    info = plsc.get_sparse_core_info()
    nc, ns, lanes = info.num_cores, info.num_subcores, info.num_lanes
    nw = nc * ns
    assert lanes == 16
    assert b % (128 * nw) == 0 and l % 8 == 0
    nj = b // 128
    ni = l // 8
    j_w = nj // nw
    ic = ni
    for cand in (5, 4, 3, 2, 1):
        if ni % cand == 0:
            ic = cand
            break
    nchunk = ni // ic

    mesh = plsc.VectorSubcoreMesh(core_axis_name="c", subcore_axis_name="s")

    @functools.partial(
        pl.kernel,
        mesh=mesh,
        out_type=jax.ShapeDtypeStruct((l, nj, 256), jnp.float32),
        scratch_types=[
            pltpu.VMEM((m, 2), jnp.float32),
            pltpu.VMEM((ic, 1, 1024), jnp.int32),
            pltpu.VMEM((8 * ic, 1, 256), jnp.float32),
        ],
        compiler_params=pltpu.CompilerParams(
            needs_layout_passes=False, use_tc_tiling_on_sc=False
        ),
    )
    def psk(zq_hbm, tab_hbm, out_hbm, tab_v, z_v, out_v):
        wid = lax.axis_index("s") * nc + lax.axis_index("c")
        pltpu.sync_copy(tab_hbm, tab_v)
        iota = lax.iota(jnp.int32, lanes)
        k0 = jnp.zeros((lanes,), jnp.int32)
        k1 = jnp.ones((lanes,), jnp.int32)
        bg16 = [iota + g * lanes for g in range(8)]
        sg16 = [iota + g * lanes + 128 for g in range(8)]

        def j_body(jj, _):
            j = wid * j_w + jj

            def chunk_body(kk, _):
                i0 = kk * ic
                pltpu.sync_copy(
                    zq_hbm.at[pl.ds(i0, ic), pl.ds(j, 1)], z_v
                )

                def i_body(i, _):
                    iv = k0 + i
                    for r in range(8):
                        lv = iv * 8 + r
                        for g in range(8):
                            rc = bg16[g] + r * 128
                            z16 = plsc.load_gather(z_v, [iv, k0, rc])
                            cc = plsc.load_gather(tab_v, [z16, k0])
                            ss = plsc.load_gather(tab_v, [z16, k1])
                            plsc.store_scatter(out_v, [lv, k0, bg16[g]], cc)
                            plsc.store_scatter(out_v, [lv, k0, sg16[g]], ss)
                    return 0

                lax.fori_loop(0, ic, i_body, 0)
                pltpu.sync_copy(
                    out_v, out_hbm.at[pl.ds(8 * i0, 8 * ic), pl.ds(j, 1)]
                )
                return 0

            lax.fori_loop(0, nchunk, chunk_body, 0)
            return 0

        lax.fori_loop(0, j_w, j_body, 0)

    return psk


def kernel(z, constellation):
    b, l = z.shape
    m = constellation.shape[0]
    zz = z.astype(jnp.int32)
    tab = constellation.astype(jnp.float32)
    zq = (
        zz.T.reshape(l // 8, 8, b // 128, 128)
        .transpose(0, 2, 1, 3)
        .reshape(l // 8, b // 128, 1024)
    )
    o = _make_psk_kernel(b, l, m)(zq, tab)
    return o.reshape(l, b // 128, 2, 128).transpose(1, 3, 0, 2).reshape(b, l, 2)
